# Initial kernel scaffold; baseline (speedup 1.0000x reference)
import jax
import jax.numpy as jnp
from jax.experimental import pallas as pl


def kernel(pts, params):
    raise NotImplementedError("write your pallas kernel here")


# speedup vs baseline: 5.6690x; 5.6690x over previous
import functools

import numpy as np
import jax
import jax.numpy as jnp
from jax import lax
from jax.experimental import pallas as pl
from jax.experimental.pallas import tpu as pltpu
from jax.experimental.pallas import tpu_sc as plsc

F32 = jnp.float32
I32 = jnp.int32
BIG = np.float32(3.0e7)
NPOINT = 10000


def _bf(x):
    return x.astype(jnp.bfloat16)


def _sumsq3(a, b, c):
    return (a * a + c * c) + b * b


def _dot(a, b):
    return lax.dot_general(_bf(a), _bf(b), (((a.ndim - 1,), (0,)), ((), ())),
                           preferred_element_type=F32)


def _fps_body(S, N, NL, xyz_ref, idx_out, cx_out, cy_out, cz_out):
    xr = xyz_ref[0]
    yr = xyz_ref[1]
    zr = xyz_ref[2]
    sub = lax.broadcasted_iota(I32, (8, NL), 0)
    lane = lax.broadcasted_iota(I32, (8, NL), 1)
    flat = sub * NL + lane
    dist0 = jnp.where(flat < N, jnp.full((8, NL), 1e10, F32),
                      jnp.full((8, NL), -1.0, F32))
    bsub = lax.broadcasted_iota(I32, (8, 128), 0)
    blane = lax.broadcasted_iota(I32, (8, 128), 1)
    bpos = bsub * 128 + blane
    buf0 = jnp.zeros((8, 128), I32)
    bf0 = jnp.zeros((8, 128), F32)

    def step(i, st):
        dist, far, bi, bx, by, bz = st
        sel = flat == far
        cx = jnp.sum(jnp.where(sel, xr, 0.0))
        cy = jnp.sum(jnp.where(sel, yr, 0.0))
        cz = jnp.sum(jnp.where(sel, zr, 0.0))
        pos = i % 1024
        lsel = bpos == pos
        bi = jnp.where(lsel, far, bi)
        bx = jnp.where(lsel, cx, bx)
        by = jnp.where(lsel, cy, by)
        bz = jnp.where(lsel, cz, bz)
        dx = xr - cx
        dy = yr - cy
        dz = zr - cz
        d = _sumsq3(dx, dy, dz)
        dist = jnp.minimum(dist, d)
        m = jnp.max(dist)
        far = jnp.min(jnp.where(dist == m, flat, jnp.int32(2 ** 30)))

        @pl.when(pos == 1023)
        def _flush():
            ro = (i // 1024) * 8
            idx_out[pl.ds(ro, 8), :] = bi
            cx_out[pl.ds(ro, 8), :] = bx
            cy_out[pl.ds(ro, 8), :] = by
            cz_out[pl.ds(ro, 8), :] = bz

        return dist, far, bi, bx, by, bz

    st = lax.fori_loop(0, S, step,
                       (dist0, jnp.int32(0), buf0, bf0, bf0, bf0))
    _, _, bi, bx, by, bz = st
    if S % 1024 != 0:
        ro = (S // 1024) * 8
        idx_out[pl.ds(ro, 8), :] = bi
        cx_out[pl.ds(ro, 8), :] = bx
        cy_out[pl.ds(ro, 8), :] = by
        cz_out[pl.ds(ro, 8), :] = bz


def _fps(xyz, S):
    N = xyz.shape[0]
    NPAD = ((N + 1023) // 1024) * 1024
    NL = NPAD // 8
    SPAD = ((S + 1023) // 1024) * 1024
    rows = SPAD // 128
    x3 = jnp.pad(xyz, ((0, NPAD - N), (0, 0))).T.reshape(3, 8, NL)
    outs = pl.pallas_call(
        functools.partial(_fps_body, S, N, NL),
        out_shape=[jax.ShapeDtypeStruct((rows, 128), I32)] +
                  [jax.ShapeDtypeStruct((rows, 128), F32)] * 3,
    )(x3)
    idx = outs[0].reshape(-1)[:S]
    new_xyz = jnp.stack([o.reshape(-1)[:S] for o in outs[1:]], axis=-1)
    return idx, new_xyz


def _select_body(N, K, r2, cb_ref, xt_ref, out_ref):
    cs = cb_ref[0]
    x0 = xt_ref[0:1, :]
    x1 = xt_ref[1:2, :]
    x2c = xt_ref[2:3, :]
    xsq = _sumsq3(x0, x1, x2c)
    dot = _dot(cs, xt_ref[...])
    c0 = cs[:, 0:1]
    c1 = cs[:, 1:2]
    c2 = cs[:, 2:3]
    cs2 = _sumsq3(c0, c1, c2)
    sqd = (cs2 - 2.0 * dot) + xsq
    col = lax.broadcasted_iota(I32, sqd.shape, 1)
    keep = jnp.logical_and(jnp.logical_not(sqd > r2), col < N)
    candv = jnp.where(keep, col.astype(F32), BIG)
    cnt = jnp.sum(keep.astype(I32), axis=1)
    tcap = jnp.minimum(jnp.max(cnt), K)
    blane = lax.broadcasted_iota(I32, (8, 128), 1)
    buf0 = jnp.full((8, 128), BIG, F32)

    def ext(k, st):
        def real(st):
            buf, prev = st
            cand = jnp.where(candv > prev, candv, BIG)
            mn = jnp.min(cand, axis=1, keepdims=True)
            buf = jnp.where(blane == k, mn, buf)
            return buf, mn

        return lax.cond(k < tcap, real, lambda s: s, st)

    buf, _ = lax.fori_loop(0, K, ext, (buf0, jnp.full((8, 1), -1.0, F32)))
    first = buf[:, 0:1]
    buf = jnp.where(buf >= BIG, first, buf)
    idxf = jnp.minimum(buf, np.float32(N - 1)).astype(I32)
    out_ref[0] = idxf[:, :K]


def _select(new_xyz, xyz, radius, K):
    S = new_xyz.shape[0]
    N = xyz.shape[0]
    NP = ((N + 127) // 128) * 128
    gs = S // 8
    cb = new_xyz.reshape(gs, 8, 3)
    xt = jnp.pad(xyz, ((0, NP - N), (0, 0))).T
    r2 = np.float32(np.float64(radius) ** 2)
    out = pl.pallas_call(
        functools.partial(_select_body, N, K, r2),
        grid=(gs,),
        in_specs=[
            pl.BlockSpec((1, 8, 3), lambda i: (i, 0, 0)),
            pl.BlockSpec((3, NP), lambda i: (0, 0)),
        ],
        out_specs=pl.BlockSpec((1, 8, K), lambda i: (i, 0, 0)),
        out_shape=jax.ShapeDtypeStruct((gs, 8, K), I32),
    )(cb, xt)
    return out.reshape(S, K)


def _sc_gather(table, idx):
    info = plsc.get_sparse_core_info()
    NC, NS = info.num_cores, info.num_subcores
    NW = NC * NS
    B = idx.shape[0]
    D = table.shape[1]
    b_per_w = B // NW
    CH = 80
    n_ch = b_per_w // CH
    mesh = plsc.VectorSubcoreMesh(core_axis_name="c", subcore_axis_name="s")

    @functools.partial(
        pl.kernel, mesh=mesh,
        out_type=jax.ShapeDtypeStruct((B, D), F32),
        compiler_params=pltpu.CompilerParams(use_tc_tiling_on_sc=False),
        scratch_types=[
            pltpu.VMEM((CH,), I32),
            pltpu.VMEM((CH, D), F32),
            pltpu.SemaphoreType.DMA,
        ],
    )
    def k(table_hbm, idx_hbm, out_hbm, idx_v, rows_v, sem):
        wid = lax.axis_index("s") * NC + lax.axis_index("c")
        base = wid * b_per_w

        def body(c, carry):
            off = base + c * CH
            pltpu.sync_copy(idx_hbm.at[pl.ds(off, CH)], idx_v)
            pltpu.async_copy(table_hbm.at[idx_v], rows_v, sem).wait()
            pltpu.sync_copy(rows_v, out_hbm.at[pl.ds(off, CH)])
            return carry

        lax.fori_loop(0, n_ch, body, 0)

    return k(table, idx)


def _b1_body(K, DP, C1, cnt, g_ref, cb_ref, w_ref, b_ref, y_ref, st_ref):
    i = pl.program_id(0)
    g = g_ref[...]
    cs = cb_ref[0]
    xyzn = g[:, :, :3] - cs[:, None, :]
    u = jnp.concatenate([xyzn, g[:, :, 3:]], axis=-1)
    y = _dot(u.reshape(8 * K, DP), w_ref[...]) + b_ref[0:1, :]
    y_ref[...] = y.reshape(8, K, C1)

    @pl.when(i == 0)
    def _init():
        st_ref[...] = jnp.zeros_like(st_ref)

    s1 = jnp.sum(y, axis=0, keepdims=True)
    s2 = jnp.sum(y * y, axis=0, keepdims=True)
    st_ref[...] += jnp.concatenate([s1, s2], axis=0)


def _b1(g, new_xyz, W1, b1, K, cnt):
    S, _, DP = g.shape
    C1 = W1.shape[0]
    gs = S // 8
    wt = jnp.zeros((DP, C1), F32).at[:W1.shape[1], :].set(W1.T)
    cb = new_xyz.reshape(gs, 8, 3)
    y, st = pl.pallas_call(
        functools.partial(_b1_body, K, DP, C1, cnt),
        grid=(gs,),
        in_specs=[
            pl.BlockSpec((8, K, DP), lambda i: (i, 0, 0)),
            pl.BlockSpec((1, 8, 3), lambda i: (i, 0, 0)),
            pl.BlockSpec((DP, C1), lambda i: (0, 0)),
            pl.BlockSpec((1, C1), lambda i: (0, 0)),
        ],
        out_specs=[
            pl.BlockSpec((8, K, C1), lambda i: (i, 0, 0)),
            pl.BlockSpec((2, C1), lambda i: (0, 0)),
        ],
        out_shape=[
            jax.ShapeDtypeStruct((S, K, C1), F32),
            jax.ShapeDtypeStruct((2, C1), F32),
        ],
    )(g, cb, wt, b1.reshape(1, C1))
    return y, st


def _norm_relu(y, st_ref, gma_ref, bta_ref, cnt):
    mean = st_ref[0:1, :] / cnt
    var = st_ref[1:2, :] / cnt - mean * mean
    xn = ((y - mean) / jnp.sqrt(var + 1e-5) * gma_ref[0:1, :]
          + bta_ref[0:1, :])
    return jnp.maximum(xn, 0.0)


def _b2_body(K, Cp, C2, cnt, y_ref, st_ref, gma_ref, bta_ref, w_ref, b_ref,
             o_ref, so_ref):
    i = pl.program_id(0)
    x = _norm_relu(y_ref[...].reshape(8 * K, Cp), st_ref, gma_ref, bta_ref,
                   cnt)
    y2 = _dot(x, w_ref[...]) + b_ref[0:1, :]
    o_ref[...] = y2.reshape(8, K, C2)

    @pl.when(i == 0)
    def _init():
        so_ref[...] = jnp.zeros_like(so_ref)

    s1 = jnp.sum(y2, axis=0, keepdims=True)
    s2 = jnp.sum(y2 * y2, axis=0, keepdims=True)
    so_ref[...] += jnp.concatenate([s1, s2], axis=0)


def _b2(y, st, gma, bta, W2, b2, K, cnt):
    S, _, Cp = y.shape
    C2 = W2.shape[0]
    gs = S // 8
    o, so = pl.pallas_call(
        functools.partial(_b2_body, K, Cp, C2, cnt),
        grid=(gs,),
        in_specs=[
            pl.BlockSpec((8, K, Cp), lambda i: (i, 0, 0)),
            pl.BlockSpec((2, Cp), lambda i: (0, 0)),
            pl.BlockSpec((1, Cp), lambda i: (0, 0)),
            pl.BlockSpec((1, Cp), lambda i: (0, 0)),
            pl.BlockSpec((Cp, C2), lambda i: (0, 0)),
            pl.BlockSpec((1, C2), lambda i: (0, 0)),
        ],
        out_specs=[
            pl.BlockSpec((8, K, C2), lambda i: (i, 0, 0)),
            pl.BlockSpec((2, C2), lambda i: (0, 0)),
        ],
        out_shape=[
            jax.ShapeDtypeStruct((S, K, C2), F32),
            jax.ShapeDtypeStruct((2, C2), F32),
        ],
    )(y, st, gma.reshape(1, Cp), bta.reshape(1, Cp), W2.T, b2.reshape(1, C2))
    return o, so


def _b4_body(K, C, cnt, fc, y_ref, st_ref, gma_ref, bta_ref, *rest):
    if fc:
        fw_ref, fb_ref, o_ref = rest
    else:
        (o_ref,) = rest
    x = _norm_relu(y_ref[...].reshape(8 * K, C), st_ref, gma_ref, bta_ref,
                   cnt).reshape(8, K, C)
    p = jnp.max(x, axis=1)
    if fc:
        p = _dot(p, fw_ref[...]) + fb_ref[0:1, :]
    o_ref[...] = p


def _b4(y, st, gma, bta, K, cnt, fc=None):
    S, _, C = y.shape
    gs = S // 8
    Co = fc[0].shape[0] if fc else C
    ins = [y, st, gma.reshape(1, C), bta.reshape(1, C)]
    in_specs = [
        pl.BlockSpec((8, K, C), lambda i: (i, 0, 0)),
        pl.BlockSpec((2, C), lambda i: (0, 0)),
        pl.BlockSpec((1, C), lambda i: (0, 0)),
        pl.BlockSpec((1, C), lambda i: (0, 0)),
    ]
    if fc:
        ins += [fc[0].T, fc[1].reshape(1, Co)]
        in_specs += [pl.BlockSpec((C, Co), lambda i: (0, 0)),
                     pl.BlockSpec((1, Co), lambda i: (0, 0))]
    out = pl.pallas_call(
        functools.partial(_b4_body, K, C, cnt, fc is not None),
        grid=(gs,),
        in_specs=in_specs,
        out_specs=pl.BlockSpec((8, Co), lambda i: (i, 0)),
        out_shape=jax.ShapeDtypeStruct((S, Co), F32),
    )(*ins)
    return out


def _sa_layer(xyz, feats, radius, K, layers, fc=None):
    N = xyz.shape[0]
    S = NPOINT
    _, new_xyz = _fps(xyz, S)
    idx = _select(new_xyz, xyz, radius, K)
    raw = jnp.concatenate([xyz, feats], axis=1)
    DP = ((raw.shape[1] + 15) // 16) * 16
    table = jnp.pad(raw, ((0, 0), (0, DP - raw.shape[1])))
    g = _sc_gather(table, idx.reshape(-1)).reshape(S, K, DP)
    cnt = np.float32(S * K)
    W1, b1, g1, be1 = layers[0]
    y, st = _b1(g, new_xyz, W1, b1, K, cnt)
    for (W, b, gm, be) in layers[1:]:
        y, st2 = _b2(y, st, g1, be1, W, b, K, cnt)
        st, g1, be1 = st2, gm, be
    p = _b4(y, st, g1, be1, K, cnt, fc=fc)
    return new_xyz, p


def kernel(pts, params):
    pts3 = pts[0]
    xyz = pts3[:3].T
    nrm = pts3[3:].T
    xyz1, p1 = _sa_layer(xyz, nrm, 0.1, 128, params['sa1'])
    xyz2, p2 = _sa_layer(xyz1, p1, 0.2, 64, params['sa2'])
    xyz3, out = _sa_layer(xyz2, p2, 0.4, 32, params['sa3'],
                          fc=(params['fc_w'], params['fc_b']))
    return (xyz3[None], out[None])
